# baseline (speedup 1.0000x reference)
import functools

import jax
import jax.numpy as jnp
from jax import lax
from jax.experimental import pallas as pl
from jax.experimental.pallas import tpu as pltpu
from jax.experimental.pallas import tpu_sc as plsc

H = 300
HP = 384
HH = HP // 2
NC, NS = 2, 16
NW = NC * NS
CHUNK = 128

NB = 1000
EB = 640


def _padw(w):
    pads = tuple((0, HP - d) if d == H else (0, 0) for d in w.shape)
    return jnp.pad(w, pads)


def _padb(b):
    return jnp.pad(b, ((0, HP - H),)).reshape(1, HP)


def _mm(a, b):
    return jnp.dot(a.astype(jnp.bfloat16), b.astype(jnp.bfloat16),
                   preferred_element_type=jnp.float32)


def _relu(a):
    return jnp.maximum(a, 0.0)


def _node_init_body(x_ref, w0, b0, w1, b1, w2, b2, wi, wo,
                    xh_out, fi_out, fj_out):
    t = _relu(_mm(x_ref[...], w0[...]) + b0[...])
    t = _relu(_mm(t, w1[...]) + b1[...])
    xh = _mm(t, w2[...]) + b2[...]
    xh_out[...] = xh
    fi_out[...] = _mm(xh, wi[...])
    fj_out[...] = _mm(xh, wo[...])


def _edge_init_body(ea_ref, w0, b0, w1, b1, w2, b2, eh_out):
    t = _relu(_mm(ea_ref[...], w0[...]) + b0[...])
    t = _relu(_mm(t, w1[...]) + b1[...])
    eh_out[...] = _mm(t, w2[...]) + b2[...]


def _edge_depth_body(scale_ref, eh_ref, g1_ref, g2_ref,
                     wl, bl, ew0, eb0, ew1, eb1, ew2, eb2,
                     nw0, nb0, nw1, nb1, nw2, nb2,
                     eh_out, m_out):
    eh = eh_ref[...]
    out = _relu(_mm(eh, wl[...]) + bl[...] + g1_ref[...] + g2_ref[...])
    t = _relu(_mm(out, ew0[...]) + eb0[...])
    t = _relu(_mm(t, ew1[...]) + eb1[...])
    eh_new = scale_ref[...] * eh + _mm(t, ew2[...]) + eb2[...]
    eh_out[...] = eh_new
    u = _relu(_mm(eh_new, nw0[...]) + nb0[...])
    u = _relu(_mm(u, nw1[...]) + nb1[...])
    m = _mm(u, nw2[...]) + nb2[...]
    m_out[0] = m[:, 0:128]
    m_out[1] = m[:, 128:256]
    m_out[2] = m[:, 256:384]


def _agg_mlp_in(agg_ref, w0g, b0):
    acc = b0[...]
    for g in range(HP // 128):
        acc = acc + _mm(agg_ref[g, 0] + agg_ref[g, 1], w0g[g])
    return _relu(acc)


def _node_depth_body(scale_ref, agg_ref, xh_ref,
                     w0g, b0, w1, b1, w2, b2, wi, wo,
                     xh_out, fi_out, fj_out):
    t = _agg_mlp_in(agg_ref, w0g, b0)
    t = _relu(_mm(t, w1[...]) + b1[...])
    xh = scale_ref[...] * xh_ref[...] + _mm(t, w2[...]) + b2[...]
    xh_out[...] = xh
    fi_out[...] = _mm(xh, wi[...])
    fj_out[...] = _mm(xh, wo[...])


def _node_final_body(scale_ref, agg_ref, xh_ref,
                     w0g, b0, w1, b1, w2, b2, xh_out):
    t = _agg_mlp_in(agg_ref, w0g, b0)
    t = _relu(_mm(t, w1[...]) + b1[...])
    xh_out[...] = scale_ref[...] * xh_ref[...] + _mm(t, w2[...]) + b2[...]


def _wspec(shape):
    nd = len(shape)
    return pl.BlockSpec(shape, lambda i: (0,) * nd)


def _rspec(rows, cols):
    return pl.BlockSpec((rows, cols), lambda i: (i, 0))


_MESH = plsc.VectorSubcoreMesh(core_axis_name="c", subcore_axis_name="s")


def _make_gather(E):
    nch = E // CHUNK
    kmax = (nch + NW - 1) // NW

    @functools.partial(
        pl.kernel,
        out_type=[jax.ShapeDtypeStruct((E, HP), jnp.float32),
                  jax.ShapeDtypeStruct((E, HP), jnp.float32)],
        mesh=_MESH,
        scratch_types=[pltpu.VMEM((CHUNK,), jnp.int32),
                       pltpu.VMEM((CHUNK, HP), jnp.float32),
                       pltpu.SemaphoreType.DMA],
    )
    def gather(fi_hbm, fj_hbm, row_hbm, col_hbm, g1_hbm, g2_hbm,
               idx_v, buf_v, sem):
        wid = lax.axis_index("s") * NC + lax.axis_index("c")

        def step(k, _):
            ch = wid + k * NW

            @pl.when(ch < nch)
            def _():
                base = ch * CHUNK
                pltpu.sync_copy(row_hbm.at[pl.ds(base, CHUNK)], idx_v)
                pltpu.async_copy(fi_hbm.at[idx_v], buf_v, sem).wait()
                pltpu.sync_copy(buf_v, g1_hbm.at[pl.ds(base, CHUNK)])
                pltpu.sync_copy(col_hbm.at[pl.ds(base, CHUNK)], idx_v)
                pltpu.async_copy(fj_hbm.at[idx_v], buf_v, sem).wait()
                pltpu.sync_copy(buf_v, g2_hbm.at[pl.ds(base, CHUNK)])

            return 0

        lax.fori_loop(0, kmax, step, 0)

    return gather


def _make_scatter(E, N):
    nch = E // CHUNK
    hch = nch // 2
    kmax = (hch + NS - 1) // NS
    ZR = 200
    ncp = N // ZR
    G = HP // 128

    @functools.partial(
        pl.kernel,
        out_type=jax.ShapeDtypeStruct((G, 2, N, 128), jnp.float32),
        mesh=_MESH,
        scratch_types=[pltpu.VMEM((CHUNK,), jnp.int32),
                       pltpu.VMEM((CHUNK, 128), jnp.float32),
                       pltpu.VMEM((ZR, 128), jnp.float32),
                       pltpu.VMEM_SHARED((N, 128), jnp.float32),
                       pltpu.SemaphoreType.DMA],
    )
    def scatter(m_hbm, col_hbm, agg_hbm, idx_v, buf_v, z_v, acc_sh, sem):
        cid = lax.axis_index("c")
        sid = lax.axis_index("s")

        def zrow(i, _):
            for j in range(128 // 16):
                z_v[i, pl.ds(j * 16, 16)] = jnp.zeros((16,), jnp.float32)
            return 0

        lax.fori_loop(0, ZR, zrow, 0)

        for g in range(G):
            def zcp(k, _):
                j = sid + k * NS

                @pl.when(j < ncp)
                def _():
                    pltpu.sync_copy(z_v, acc_sh.at[pl.ds(j * ZR, ZR)])

                return 0

            lax.fori_loop(0, (ncp + NS - 1) // NS, zcp, 0)
            plsc.subcore_barrier()

            def step(k, _):
                ch = cid * hch + sid + k * NS

                @pl.when(sid + k * NS < hch)
                def _():
                    base = ch * CHUNK
                    pltpu.sync_copy(col_hbm.at[pl.ds(base, CHUNK)], idx_v)
                    pltpu.sync_copy(m_hbm.at[g, pl.ds(base, CHUNK)], buf_v)
                    pltpu.sync_copy(buf_v, acc_sh.at[idx_v], add=True)

                return 0

            lax.fori_loop(0, kmax, step, 0)
            plsc.subcore_barrier()

            def wcp(k, _):
                j = sid + k * NS

                @pl.when(j < ncp)
                def _():
                    pltpu.sync_copy(acc_sh.at[pl.ds(j * ZR, ZR)],
                                    agg_hbm.at[g, cid, pl.ds(j * ZR, ZR)])

                return 0

            lax.fori_loop(0, (ncp + NS - 1) // NS, wcp, 0)
            plsc.subcore_barrier()

    return scatter


def kernel(x, edge_index, edge_attr, params):
    N, ND = x.shape
    E, ED = edge_attr.shape
    row = edge_index[0]
    col = edge_index[1]

    p = params
    escale = jnp.full((1, HP), 1.0 + p["edge_eps"][0], jnp.float32)
    nscale = jnp.full((1, HP), 1.0 + p["node_eps"][0], jnp.float32)

    ni = {k: _padw(v) if v.ndim == 2 else _padb(v)
          for k, v in p["node_init"].items()}
    ei = {k: _padw(v) if v.ndim == 2 else _padb(v)
          for k, v in p["edge_init"].items()}
    em = {k: _padw(v) if v.ndim == 2 else _padb(v)
          for k, v in p["edge_mlp"].items()}
    n1 = {k: _padw(v) if v.ndim == 2 else _padb(v)
          for k, v in p["node_mlp1"].items()}
    n2 = {k: _padw(v) if v.ndim == 2 else _padb(v)
          for k, v in p["node_mlp2"].items()}
    wl = _padw(p["edge_lin_W"])
    bl = _padb(p["edge_lin_b"])
    wi = _padw(p["node_in_W"])
    wo = _padw(p["node_out_W"])
    n2w0g = n2["W0"].reshape(HP // 128, 128, HP)

    fdt = jnp.float32
    ngrid = (N // NB,)
    egrid = (E // EB,)

    node_init = pl.pallas_call(
        _node_init_body,
        grid=ngrid,
        in_specs=[_rspec(NB, ND),
                  _wspec((ND, HP)), _wspec((1, HP)),
                  _wspec((HP, HP)), _wspec((1, HP)),
                  _wspec((HP, HP)), _wspec((1, HP)),
                  _wspec((HP, HP)), _wspec((HP, HP))],
        out_specs=[_rspec(NB, HP)] * 3,
        out_shape=[jax.ShapeDtypeStruct((N, HP), fdt)] * 3,
    )
    xh, f_i, f_j = node_init(x, ni["W0"], ni["b0"], ni["W1"], ni["b1"],
                             ni["W2"], ni["b2"], wi, wo)

    edge_init = pl.pallas_call(
        _edge_init_body,
        grid=egrid,
        in_specs=[_rspec(EB, ED),
                  _wspec((ED, HP)), _wspec((1, HP)),
                  _wspec((HP, HP)), _wspec((1, HP)),
                  _wspec((HP, HP)), _wspec((1, HP))],
        out_specs=_rspec(EB, HP),
        out_shape=jax.ShapeDtypeStruct((E, HP), fdt),
    )
    eh = edge_init(edge_attr, ei["W0"], ei["b0"], ei["W1"], ei["b1"],
                   ei["W2"], ei["b2"])

    gather = _make_gather(E)
    scatter = _make_scatter(E, N)

    edge_depth = pl.pallas_call(
        _edge_depth_body,
        grid=egrid,
        in_specs=[_wspec((1, HP)),
                  _rspec(EB, HP), _rspec(EB, HP), _rspec(EB, HP),
                  _wspec((HP, HP)), _wspec((1, HP)),
                  _wspec((HP, HP)), _wspec((1, HP)),
                  _wspec((HP, HP)), _wspec((1, HP)),
                  _wspec((HP, HP)), _wspec((1, HP)),
                  _wspec((HP, HP)), _wspec((1, HP)),
                  _wspec((HP, HP)), _wspec((1, HP)),
                  _wspec((HP, HP)), _wspec((1, HP))],
        out_specs=[_rspec(EB, HP),
                   pl.BlockSpec((3, EB, 128), lambda i: (0, i, 0))],
        out_shape=[jax.ShapeDtypeStruct((E, HP), fdt),
                   jax.ShapeDtypeStruct((3, E, 128), fdt)],
    )

    node_depth = pl.pallas_call(
        _node_depth_body,
        grid=ngrid,
        in_specs=[_wspec((1, HP)),
                  pl.BlockSpec((3, 2, NB, 128), lambda i: (0, 0, i, 0)),
                  _rspec(NB, HP),
                  _wspec((3, 128, HP)), _wspec((1, HP)),
                  _wspec((HP, HP)), _wspec((1, HP)),
                  _wspec((HP, HP)), _wspec((1, HP)),
                  _wspec((HP, HP)), _wspec((HP, HP))],
        out_specs=[_rspec(NB, HP)] * 3,
        out_shape=[jax.ShapeDtypeStruct((N, HP), fdt)] * 3,
    )

    node_final = pl.pallas_call(
        _node_final_body,
        grid=ngrid,
        in_specs=[_wspec((1, HP)),
                  pl.BlockSpec((3, 2, NB, 128), lambda i: (0, 0, i, 0)),
                  _rspec(NB, HP),
                  _wspec((3, 128, HP)), _wspec((1, HP)),
                  _wspec((HP, HP)), _wspec((1, HP)),
                  _wspec((HP, HP)), _wspec((1, HP))],
        out_specs=_rspec(NB, HP),
        out_shape=jax.ShapeDtypeStruct((N, HP), fdt),
    )

    for d in range(3):
        g1, g2 = gather(f_i, f_j, row, col)
        eh, m = edge_depth(escale, eh, g1, g2,
                           wl, bl, em["W0"], em["b0"], em["W1"], em["b1"],
                           em["W2"], em["b2"], n1["W0"], n1["b0"], n1["W1"],
                           n1["b1"], n1["W2"], n1["b2"])
        agg = scatter(m, col)
        if d < 2:
            xh, f_i, f_j = node_depth(nscale, agg, xh,
                                      n2w0g, n2["b0"], n2["W1"],
                                      n2["b1"], n2["W2"], n2["b2"], wi, wo)
        else:
            xh = node_final(nscale, agg, xh,
                            n2w0g, n2["b0"], n2["W1"], n2["b1"],
                            n2["W2"], n2["b2"])

    return xh[:, :H], eh[:, :H]


# speedup vs baseline: 1.0293x; 1.0293x over previous
import functools

import jax
import jax.numpy as jnp
from jax import lax
from jax.experimental import pallas as pl
from jax.experimental.pallas import tpu as pltpu
from jax.experimental.pallas import tpu_sc as plsc

H = 300
HP = 384
HH = HP // 2
NC, NS = 2, 16
NW = NC * NS
CHUNK = 128

NB = 1000
EB = 640


def _padw(w):
    pads = tuple((0, HP - d) if d == H else (0, 0) for d in w.shape)
    return jnp.pad(w, pads)


def _padb(b):
    return jnp.pad(b, ((0, HP - H),)).reshape(1, HP)


def _mm(a, b):
    return jnp.dot(a.astype(jnp.bfloat16), b.astype(jnp.bfloat16),
                   preferred_element_type=jnp.float32)


def _relu(a):
    return jnp.maximum(a, 0.0)


def _node_init_body(x_ref, w0, b0, w1, b1, w2, b2, wi, wo,
                    xh_out, fi_out, fj_out):
    t = _relu(_mm(x_ref[...], w0[...]) + b0[...])
    t = _relu(_mm(t, w1[...]) + b1[...])
    xh = _mm(t, w2[...]) + b2[...]
    xh_out[...] = xh
    fi_out[...] = _mm(xh, wi[...])
    fj_out[...] = _mm(xh, wo[...])


def _edge_init_body(ea_ref, w0, b0, w1, b1, w2, b2, eh_out):
    t = _relu(_mm(ea_ref[...], w0[...]) + b0[...])
    t = _relu(_mm(t, w1[...]) + b1[...])
    eh_out[...] = _mm(t, w2[...]) + b2[...]


def _make_edge_depth_body(out_w):
    def body(scale_ref, eh_ref, g_ref,
             wl, bl, ew0, eb0, ew1, eb1, ew2, eb2,
             nw0, nb0, nw1, nb1, nw2, nb2,
             eh_out, m_out):
        eh = eh_ref[...]
        out = _relu(_mm(eh, wl[...]) + bl[...] + g_ref[...])
        t = _relu(_mm(out, ew0[...]) + eb0[...])
        t = _relu(_mm(t, ew1[...]) + eb1[...])
        eh_new = scale_ref[...] * eh + _mm(t, ew2[...]) + eb2[...]
        eh_out[...] = eh_new[:, :out_w]
        u = _relu(_mm(eh_new, nw0[...]) + nb0[...])
        u = _relu(_mm(u, nw1[...]) + nb1[...])
        m = _mm(u, nw2[...]) + nb2[...]
        m_out[0] = m[:, 0:128]
        m_out[1] = m[:, 128:256]
        m_out[2] = m[:, 256:384]

    return body


def _agg_mlp_in(agg_ref, w0g, b0):
    acc = b0[...]
    for g in range(HP // 128):
        acc = acc + _mm(agg_ref[g, 0] + agg_ref[g, 1], w0g[g])
    return _relu(acc)


def _node_depth_body(scale_ref, agg_ref, xh_ref,
                     w0g, b0, w1, b1, w2, b2, wi, wo,
                     xh_out, fi_out, fj_out):
    t = _agg_mlp_in(agg_ref, w0g, b0)
    t = _relu(_mm(t, w1[...]) + b1[...])
    xh = scale_ref[...] * xh_ref[...] + _mm(t, w2[...]) + b2[...]
    xh_out[...] = xh
    fi_out[...] = _mm(xh, wi[...])
    fj_out[...] = _mm(xh, wo[...])


def _node_final_body(scale_ref, agg_ref, xh_ref,
                     w0g, b0, w1, b1, w2, b2, xh_out):
    t = _agg_mlp_in(agg_ref, w0g, b0)
    t = _relu(_mm(t, w1[...]) + b1[...])
    xh = scale_ref[...] * xh_ref[...] + _mm(t, w2[...]) + b2[...]
    xh_out[...] = xh[:, :H]


def _wspec(shape):
    nd = len(shape)
    return pl.BlockSpec(shape, lambda i: (0,) * nd)


def _rspec(rows, cols):
    return pl.BlockSpec((rows, cols), lambda i: (i, 0))


_MESH = plsc.VectorSubcoreMesh(core_axis_name="c", subcore_axis_name="s")


def _make_gather(E):
    nch = E // CHUNK
    kmax = (nch + NW - 1) // NW

    @functools.partial(
        pl.kernel,
        out_type=jax.ShapeDtypeStruct((E, HP), jnp.float32),
        mesh=_MESH,
        scratch_types=[pltpu.VMEM((CHUNK,), jnp.int32),
                       pltpu.VMEM((CHUNK,), jnp.int32),
                       pltpu.VMEM((CHUNK, HP), jnp.float32),
                       pltpu.VMEM((CHUNK, HP), jnp.float32),
                       pltpu.SemaphoreType.DMA,
                       pltpu.SemaphoreType.DMA],
    )
    def gather(fi_hbm, fj_hbm, row_hbm, col_hbm, g_hbm,
               idxa_v, idxb_v, bufa_v, bufb_v, sema, semb):
        wid = lax.axis_index("s") * NC + lax.axis_index("c")

        def step(k, _):
            ch = wid + k * NW

            @pl.when(ch < nch)
            def _():
                base = ch * CHUNK
                pltpu.sync_copy(row_hbm.at[pl.ds(base, CHUNK)], idxa_v)
                pltpu.sync_copy(col_hbm.at[pl.ds(base, CHUNK)], idxb_v)
                ca = pltpu.async_copy(fi_hbm.at[idxa_v], bufa_v, sema)
                cb = pltpu.async_copy(fj_hbm.at[idxb_v], bufb_v, semb)
                ca.wait()
                cb.wait()

                def addrow(r, _):
                    for j in range(HP // 16):
                        plsc.addupdate(bufa_v.at[r, pl.ds(j * 16, 16)],
                                       bufb_v[r, pl.ds(j * 16, 16)])
                    return 0

                lax.fori_loop(0, CHUNK, addrow, 0)
                pltpu.sync_copy(bufa_v, g_hbm.at[pl.ds(base, CHUNK)])

            return 0

        lax.fori_loop(0, kmax, step, 0)

    return gather


def _make_scatter(E, N):
    nch = E // CHUNK
    hch = nch // 2
    kmax = (hch + NS - 1) // NS
    ZR = 200
    ncp = N // ZR
    G = HP // 128

    @functools.partial(
        pl.kernel,
        out_type=jax.ShapeDtypeStruct((G, 2, N, 128), jnp.float32),
        mesh=_MESH,
        scratch_types=[pltpu.VMEM((CHUNK,), jnp.int32),
                       pltpu.VMEM((CHUNK, 128), jnp.float32),
                       pltpu.VMEM((ZR, 128), jnp.float32),
                       pltpu.VMEM_SHARED((N, 128), jnp.float32),
                       pltpu.SemaphoreType.DMA],
    )
    def scatter(m_hbm, col_hbm, agg_hbm, idx_v, buf_v, z_v, acc_sh, sem):
        cid = lax.axis_index("c")
        sid = lax.axis_index("s")

        def zrow(i, _):
            for j in range(128 // 16):
                z_v[i, pl.ds(j * 16, 16)] = jnp.zeros((16,), jnp.float32)
            return 0

        lax.fori_loop(0, ZR, zrow, 0)

        for g in range(G):
            def zcp(k, _):
                j = sid + k * NS

                @pl.when(j < ncp)
                def _():
                    pltpu.sync_copy(z_v, acc_sh.at[pl.ds(j * ZR, ZR)])

                return 0

            lax.fori_loop(0, (ncp + NS - 1) // NS, zcp, 0)
            plsc.subcore_barrier()

            def step(k, _):
                ch = cid * hch + sid + k * NS

                @pl.when(sid + k * NS < hch)
                def _():
                    base = ch * CHUNK
                    pltpu.sync_copy(col_hbm.at[pl.ds(base, CHUNK)], idx_v)
                    pltpu.sync_copy(m_hbm.at[g, pl.ds(base, CHUNK)], buf_v)
                    pltpu.sync_copy(buf_v, acc_sh.at[idx_v], add=True)

                return 0

            lax.fori_loop(0, kmax, step, 0)
            plsc.subcore_barrier()

            def wcp(k, _):
                j = sid + k * NS

                @pl.when(j < ncp)
                def _():
                    pltpu.sync_copy(acc_sh.at[pl.ds(j * ZR, ZR)],
                                    agg_hbm.at[g, cid, pl.ds(j * ZR, ZR)])

                return 0

            lax.fori_loop(0, (ncp + NS - 1) // NS, wcp, 0)
            plsc.subcore_barrier()

    return scatter


def kernel(x, edge_index, edge_attr, params):
    N, ND = x.shape
    E, ED = edge_attr.shape
    row = edge_index[0]
    col = edge_index[1]

    p = params
    escale = jnp.full((1, HP), 1.0 + p["edge_eps"][0], jnp.float32)
    nscale = jnp.full((1, HP), 1.0 + p["node_eps"][0], jnp.float32)

    ni = {k: _padw(v) if v.ndim == 2 else _padb(v)
          for k, v in p["node_init"].items()}
    ei = {k: _padw(v) if v.ndim == 2 else _padb(v)
          for k, v in p["edge_init"].items()}
    em = {k: _padw(v) if v.ndim == 2 else _padb(v)
          for k, v in p["edge_mlp"].items()}
    n1 = {k: _padw(v) if v.ndim == 2 else _padb(v)
          for k, v in p["node_mlp1"].items()}
    n2 = {k: _padw(v) if v.ndim == 2 else _padb(v)
          for k, v in p["node_mlp2"].items()}
    wl = _padw(p["edge_lin_W"])
    bl = _padb(p["edge_lin_b"])
    wi = _padw(p["node_in_W"])
    wo = _padw(p["node_out_W"])
    n2w0g = n2["W0"].reshape(HP // 128, 128, HP)

    fdt = jnp.float32
    ngrid = (N // NB,)
    egrid = (E // EB,)

    node_init = pl.pallas_call(
        _node_init_body,
        grid=ngrid,
        in_specs=[_rspec(NB, ND),
                  _wspec((ND, HP)), _wspec((1, HP)),
                  _wspec((HP, HP)), _wspec((1, HP)),
                  _wspec((HP, HP)), _wspec((1, HP)),
                  _wspec((HP, HP)), _wspec((HP, HP))],
        out_specs=[_rspec(NB, HP)] * 3,
        out_shape=[jax.ShapeDtypeStruct((N, HP), fdt),
                   jax.ShapeDtypeStruct((N, HP), fdt),
                   jax.ShapeDtypeStruct((N, HP), fdt)],
    )
    xh, f_i, f_j = node_init(x, ni["W0"], ni["b0"], ni["W1"], ni["b1"],
                             ni["W2"], ni["b2"], wi, wo)

    edge_init = pl.pallas_call(
        _edge_init_body,
        grid=egrid,
        in_specs=[_rspec(EB, ED),
                  _wspec((ED, HP)), _wspec((1, HP)),
                  _wspec((HP, HP)), _wspec((1, HP)),
                  _wspec((HP, HP)), _wspec((1, HP))],
        out_specs=_rspec(EB, HP),
        out_shape=jax.ShapeDtypeStruct((E, HP), fdt),
    )
    eh = edge_init(edge_attr, ei["W0"], ei["b0"], ei["W1"], ei["b1"],
                   ei["W2"], ei["b2"])

    gather = _make_gather(E)
    scatter = _make_scatter(E, N)

    edge_specs = dict(
        grid=egrid,
        in_specs=[_wspec((1, HP)),
                  _rspec(EB, HP), _rspec(EB, HP),
                  _wspec((HP, HP)), _wspec((1, HP)),
                  _wspec((HP, HP)), _wspec((1, HP)),
                  _wspec((HP, HP)), _wspec((1, HP)),
                  _wspec((HP, HP)), _wspec((1, HP)),
                  _wspec((HP, HP)), _wspec((1, HP)),
                  _wspec((HP, HP)), _wspec((1, HP)),
                  _wspec((HP, HP)), _wspec((1, HP))],
    )
    edge_depth = pl.pallas_call(
        _make_edge_depth_body(HP),
        out_specs=[_rspec(EB, HP),
                   pl.BlockSpec((3, EB, 128), lambda i: (0, i, 0))],
        out_shape=[jax.ShapeDtypeStruct((E, HP), fdt),
                   jax.ShapeDtypeStruct((3, E, 128), fdt)],
        **edge_specs,
    )
    edge_depth_final = pl.pallas_call(
        _make_edge_depth_body(H),
        out_specs=[_rspec(EB, H),
                   pl.BlockSpec((3, EB, 128), lambda i: (0, i, 0))],
        out_shape=[jax.ShapeDtypeStruct((E, H), fdt),
                   jax.ShapeDtypeStruct((3, E, 128), fdt)],
        **edge_specs,
    )

    node_depth = pl.pallas_call(
        _node_depth_body,
        grid=ngrid,
        in_specs=[_wspec((1, HP)),
                  pl.BlockSpec((3, 2, NB, 128), lambda i: (0, 0, i, 0)),
                  _rspec(NB, HP),
                  _wspec((3, 128, HP)), _wspec((1, HP)),
                  _wspec((HP, HP)), _wspec((1, HP)),
                  _wspec((HP, HP)), _wspec((1, HP)),
                  _wspec((HP, HP)), _wspec((HP, HP))],
        out_specs=[_rspec(NB, HP)] * 3,
        out_shape=[jax.ShapeDtypeStruct((N, HP), fdt),
                   jax.ShapeDtypeStruct((N, HP), fdt),
                   jax.ShapeDtypeStruct((N, HP), fdt)],
    )

    node_final = pl.pallas_call(
        _node_final_body,
        grid=ngrid,
        in_specs=[_wspec((1, HP)),
                  pl.BlockSpec((3, 2, NB, 128), lambda i: (0, 0, i, 0)),
                  _rspec(NB, HP),
                  _wspec((3, 128, HP)), _wspec((1, HP)),
                  _wspec((HP, HP)), _wspec((1, HP)),
                  _wspec((HP, HP)), _wspec((1, HP))],
        out_specs=_rspec(NB, H),
        out_shape=jax.ShapeDtypeStruct((N, H), fdt),
    )

    for d in range(3):
        g = gather(f_i, f_j, row, col)
        ed = edge_depth if d < 2 else edge_depth_final
        eh, m = ed(escale, eh, g,
                   wl, bl, em["W0"], em["b0"], em["W1"], em["b1"],
                   em["W2"], em["b2"], n1["W0"], n1["b0"], n1["W1"],
                   n1["b1"], n1["W2"], n1["b2"])
        agg = scatter(m, col)
        if d < 2:
            xh, f_i, f_j = node_depth(nscale, agg, xh,
                                      n2w0g, n2["b0"], n2["W1"],
                                      n2["b1"], n2["W2"], n2["b2"], wi, wo)
        else:
            xh = node_final(nscale, agg, xh,
                            n2w0g, n2["b0"], n2["W1"], n2["b1"],
                            n2["W2"], n2["b2"])

    return xh, eh


# speedup vs baseline: 1.1766x; 1.1431x over previous
import functools

import jax
import jax.numpy as jnp
from jax import lax
from jax.experimental import pallas as pl
from jax.experimental.pallas import tpu as pltpu
from jax.experimental.pallas import tpu_sc as plsc

H = 300
HP = 384
HH = HP // 2
NC, NS = 2, 16
NW = NC * NS
CHUNK = 128

NB = 1000
EB = 640


def _padw(w):
    pads = tuple((0, HP - d) if d == H else (0, 0) for d in w.shape)
    return jnp.pad(w, pads)


def _padb(b):
    return jnp.pad(b, ((0, HP - H),)).reshape(1, HP)


def _mm(a, b):
    return jnp.dot(a.astype(jnp.bfloat16), b.astype(jnp.bfloat16),
                   preferred_element_type=jnp.float32)


def _relu(a):
    return jnp.maximum(a, 0.0)


def _node_init_body(x_ref, w0, b0, w1, b1, w2, b2, wi, wo,
                    xh_out, fi_out, fj_out):
    t = _relu(_mm(x_ref[...], w0[...]) + b0[...])
    t = _relu(_mm(t, w1[...]) + b1[...])
    xh = _mm(t, w2[...]) + b2[...]
    xh_out[...] = xh
    fi_out[...] = _mm(xh, wi[...])
    fj_out[...] = _mm(xh, wo[...])


def _edge_init_body(ea_ref, w0, b0, w1, b1, w2, b2, eh_out):
    t = _relu(_mm(ea_ref[...], w0[...]) + b0[...])
    t = _relu(_mm(t, w1[...]) + b1[...])
    eh_out[...] = _mm(t, w2[...]) + b2[...]


def _make_edge_depth_body(out_w):
    def body(scale_ref, eh_ref, g_ref,
             wl, bl, ew0, eb0, ew1, eb1, ew2, eb2,
             nw0, nb0, nw1, nb1, nw2, nb2,
             eh_out, m_out):
        eh = eh_ref[...]
        out = _relu(_mm(eh, wl[...]) + bl[...] + g_ref[...])
        t = _relu(_mm(out, ew0[...]) + eb0[...])
        t = _relu(_mm(t, ew1[...]) + eb1[...])
        eh_new = scale_ref[...] * eh + _mm(t, ew2[...]) + eb2[...]
        eh_out[...] = eh_new[:, :out_w]
        u = _relu(_mm(eh_new, nw0[...]) + nb0[...])
        u = _relu(_mm(u, nw1[...]) + nb1[...])
        m = _mm(u, nw2[...]) + nb2[...]
        m_out[0] = m[:, 0:128]
        m_out[1] = m[:, 128:256]
        m_out[2] = m[:, 256:384]

    return body


def _agg_mlp_in(agg_ref, w0g, b0):
    acc = b0[...]
    for g in range(HP // 128):
        acc = acc + _mm(agg_ref[g, 0] + agg_ref[g, 1], w0g[g])
    return _relu(acc)


def _node_depth_body(scale_ref, agg_ref, xh_ref,
                     w0g, b0, w1, b1, w2, b2, wi, wo,
                     xh_out, fi_out, fj_out):
    t = _agg_mlp_in(agg_ref, w0g, b0)
    t = _relu(_mm(t, w1[...]) + b1[...])
    xh = scale_ref[...] * xh_ref[...] + _mm(t, w2[...]) + b2[...]
    xh_out[...] = xh
    fi_out[...] = _mm(xh, wi[...])
    fj_out[...] = _mm(xh, wo[...])


def _node_final_body(scale_ref, agg_ref, xh_ref,
                     w0g, b0, w1, b1, w2, b2, xh_out):
    t = _agg_mlp_in(agg_ref, w0g, b0)
    t = _relu(_mm(t, w1[...]) + b1[...])
    xh = scale_ref[...] * xh_ref[...] + _mm(t, w2[...]) + b2[...]
    xh_out[...] = xh[:, :H]


def _wspec(shape):
    nd = len(shape)
    return pl.BlockSpec(shape, lambda i: (0,) * nd)


def _rspec(rows, cols):
    return pl.BlockSpec((rows, cols), lambda i: (i, 0))


_MESH = plsc.VectorSubcoreMesh(core_axis_name="c", subcore_axis_name="s")


CG = 40


def _make_gather(E):
    per = E // NW
    nk = per // CG

    @functools.partial(
        pl.kernel,
        out_type=jax.ShapeDtypeStruct((E, HP), jnp.float32),
        mesh=_MESH,
        scratch_types=[pltpu.VMEM((per,), jnp.int32),
                       pltpu.VMEM((per,), jnp.int32),
                       pltpu.VMEM((CG, HP), jnp.float32),
                       pltpu.VMEM((CG, HP), jnp.float32),
                       pltpu.VMEM((CG, HP), jnp.float32),
                       pltpu.VMEM((CG, HP), jnp.float32),
                       pltpu.SemaphoreType.DMA,
                       pltpu.SemaphoreType.DMA,
                       pltpu.SemaphoreType.DMA,
                       pltpu.SemaphoreType.DMA],
    )
    def gather(fi_hbm, fj_hbm, row_hbm, col_hbm, g_hbm,
               idxa_v, idxb_v, ba0, ba1, bb0, bb1, sa0, sa1, sb0, sb1):
        wid = lax.axis_index("s") * NC + lax.axis_index("c")
        e0 = wid * per
        pltpu.sync_copy(row_hbm.at[pl.ds(e0, per)], idxa_v)
        pltpu.sync_copy(col_hbm.at[pl.ds(e0, per)], idxb_v)
        bufs = ((ba0, bb0, sa0, sb0), (ba1, bb1, sa1, sb1))

        def issue(k, p):
            ba, bb, sa, sb = bufs[p]
            s = pl.ds(k * CG, CG)
            pltpu.async_copy(fi_hbm.at[idxa_v.at[s]], ba, sa)
            pltpu.async_copy(fj_hbm.at[idxb_v.at[s]], bb, sb)

        def consume(k, p):
            ba, bb, sa, sb = bufs[p]
            s = pl.ds(k * CG, CG)
            pltpu.make_async_copy(fi_hbm.at[idxa_v.at[s]], ba, sa).wait()
            pltpu.make_async_copy(fj_hbm.at[idxb_v.at[s]], bb, sb).wait()

            def addrow(r, _):
                for j in range(HP // 16):
                    plsc.addupdate(ba.at[r, pl.ds(j * 16, 16)],
                                   bb[r, pl.ds(j * 16, 16)])
                return 0

            lax.fori_loop(0, CG, addrow, 0)
            pltpu.sync_copy(ba, g_hbm.at[pl.ds(e0 + k * CG, CG)])

        issue(0, 0)

        def step(kk, _):
            k0 = 2 * kk
            issue(k0 + 1, 1)
            consume(k0, 0)
            issue(k0 + 2, 0)
            consume(k0 + 1, 1)
            return 0

        lax.fori_loop(0, (nk - 1) // 2, step, 0)
        consume(nk - 1, 0)

    return gather


def _make_scatter(E, N):
    per = E // NW
    nk = per // CG
    ZR = 200
    ncp = N // ZR
    G = HP // 128

    @functools.partial(
        pl.kernel,
        out_type=jax.ShapeDtypeStruct((G, 2, N, 128), jnp.float32),
        mesh=_MESH,
        scratch_types=[pltpu.VMEM((CG,), jnp.int32),
                       pltpu.VMEM((CG,), jnp.int32),
                       pltpu.VMEM((CG, 128), jnp.float32),
                       pltpu.VMEM((CG, 128), jnp.float32),
                       pltpu.VMEM_SHARED((N, 128), jnp.float32),
                       pltpu.SemaphoreType.DMA,
                       pltpu.SemaphoreType.DMA,
                       pltpu.SemaphoreType.DMA,
                       pltpu.SemaphoreType.DMA],
    )
    def scatter(m_hbm, col_hbm, z_hbm, agg_hbm, ib0, ib1, mb0, mb1, acc_sh,
                si0, si1, sm0, sm1):
        cid = lax.axis_index("c")
        sid = lax.axis_index("s")
        e0 = (cid * NS + sid) * per
        mbufs = ((ib0, mb0, si0, sm0), (ib1, mb1, si1, sm1))

        for g in range(G):
            def zcp(k, _):
                j = sid + k * NS

                @pl.when(j < ncp)
                def _():
                    pltpu.sync_copy(z_hbm, acc_sh.at[pl.ds(j * ZR, ZR)])

                return 0

            lax.fori_loop(0, (ncp + NS - 1) // NS, zcp, 0)
            plsc.subcore_barrier()

            def issue(k, p):
                ib, mb, si, sm = mbufs[p]
                pltpu.async_copy(col_hbm.at[pl.ds(e0 + k * CG, CG)], ib, si)
                pltpu.async_copy(m_hbm.at[g, pl.ds(e0 + k * CG, CG)], mb, sm)

            def consume(k, p):
                ib, mb, si, sm = mbufs[p]
                pltpu.make_async_copy(
                    col_hbm.at[pl.ds(e0 + k * CG, CG)], ib, si).wait()
                pltpu.make_async_copy(
                    m_hbm.at[g, pl.ds(e0 + k * CG, CG)], mb, sm).wait()
                pltpu.sync_copy(mb, acc_sh.at[ib], add=True)

            issue(0, 0)

            def step(kk, _):
                k0 = 2 * kk
                issue(k0 + 1, 1)
                consume(k0, 0)
                issue(k0 + 2, 0)
                consume(k0 + 1, 1)
                return 0

            lax.fori_loop(0, (nk - 1) // 2, step, 0)
            consume(nk - 1, 0)
            plsc.subcore_barrier()

            def wcp(k, _):
                j = sid + k * NS

                @pl.when(j < ncp)
                def _():
                    pltpu.sync_copy(acc_sh.at[pl.ds(j * ZR, ZR)],
                                    agg_hbm.at[g, cid, pl.ds(j * ZR, ZR)])

                return 0

            lax.fori_loop(0, (ncp + NS - 1) // NS, wcp, 0)
            plsc.subcore_barrier()

    return scatter


def kernel(x, edge_index, edge_attr, params):
    N, ND = x.shape
    E, ED = edge_attr.shape
    row = edge_index[0]
    col = edge_index[1]
    zeros_zr = jnp.zeros((200, 128), jnp.float32)

    p = params
    escale = jnp.full((1, HP), 1.0 + p["edge_eps"][0], jnp.float32)
    nscale = jnp.full((1, HP), 1.0 + p["node_eps"][0], jnp.float32)

    ni = {k: _padw(v) if v.ndim == 2 else _padb(v)
          for k, v in p["node_init"].items()}
    ei = {k: _padw(v) if v.ndim == 2 else _padb(v)
          for k, v in p["edge_init"].items()}
    em = {k: _padw(v) if v.ndim == 2 else _padb(v)
          for k, v in p["edge_mlp"].items()}
    n1 = {k: _padw(v) if v.ndim == 2 else _padb(v)
          for k, v in p["node_mlp1"].items()}
    n2 = {k: _padw(v) if v.ndim == 2 else _padb(v)
          for k, v in p["node_mlp2"].items()}
    wl = _padw(p["edge_lin_W"])
    bl = _padb(p["edge_lin_b"])
    wi = _padw(p["node_in_W"])
    wo = _padw(p["node_out_W"])
    n2w0g = n2["W0"].reshape(HP // 128, 128, HP)

    fdt = jnp.float32
    ngrid = (N // NB,)
    egrid = (E // EB,)

    node_init = pl.pallas_call(
        _node_init_body,
        grid=ngrid,
        in_specs=[_rspec(NB, ND),
                  _wspec((ND, HP)), _wspec((1, HP)),
                  _wspec((HP, HP)), _wspec((1, HP)),
                  _wspec((HP, HP)), _wspec((1, HP)),
                  _wspec((HP, HP)), _wspec((HP, HP))],
        out_specs=[_rspec(NB, HP)] * 3,
        out_shape=[jax.ShapeDtypeStruct((N, HP), fdt),
                   jax.ShapeDtypeStruct((N, HP), fdt),
                   jax.ShapeDtypeStruct((N, HP), fdt)],
    )
    xh, f_i, f_j = node_init(x, ni["W0"], ni["b0"], ni["W1"], ni["b1"],
                             ni["W2"], ni["b2"], wi, wo)

    edge_init = pl.pallas_call(
        _edge_init_body,
        grid=egrid,
        in_specs=[_rspec(EB, ED),
                  _wspec((ED, HP)), _wspec((1, HP)),
                  _wspec((HP, HP)), _wspec((1, HP)),
                  _wspec((HP, HP)), _wspec((1, HP))],
        out_specs=_rspec(EB, HP),
        out_shape=jax.ShapeDtypeStruct((E, HP), fdt),
    )
    eh = edge_init(edge_attr, ei["W0"], ei["b0"], ei["W1"], ei["b1"],
                   ei["W2"], ei["b2"])

    gather = _make_gather(E)
    scatter = _make_scatter(E, N)

    edge_specs = dict(
        grid=egrid,
        in_specs=[_wspec((1, HP)),
                  _rspec(EB, HP), _rspec(EB, HP),
                  _wspec((HP, HP)), _wspec((1, HP)),
                  _wspec((HP, HP)), _wspec((1, HP)),
                  _wspec((HP, HP)), _wspec((1, HP)),
                  _wspec((HP, HP)), _wspec((1, HP)),
                  _wspec((HP, HP)), _wspec((1, HP)),
                  _wspec((HP, HP)), _wspec((1, HP)),
                  _wspec((HP, HP)), _wspec((1, HP))],
    )
    edge_depth = pl.pallas_call(
        _make_edge_depth_body(HP),
        out_specs=[_rspec(EB, HP),
                   pl.BlockSpec((3, EB, 128), lambda i: (0, i, 0))],
        out_shape=[jax.ShapeDtypeStruct((E, HP), fdt),
                   jax.ShapeDtypeStruct((3, E, 128), fdt)],
        **edge_specs,
    )
    edge_depth_final = pl.pallas_call(
        _make_edge_depth_body(H),
        out_specs=[_rspec(EB, H),
                   pl.BlockSpec((3, EB, 128), lambda i: (0, i, 0))],
        out_shape=[jax.ShapeDtypeStruct((E, H), fdt),
                   jax.ShapeDtypeStruct((3, E, 128), fdt)],
        **edge_specs,
    )

    node_depth = pl.pallas_call(
        _node_depth_body,
        grid=ngrid,
        in_specs=[_wspec((1, HP)),
                  pl.BlockSpec((3, 2, NB, 128), lambda i: (0, 0, i, 0)),
                  _rspec(NB, HP),
                  _wspec((3, 128, HP)), _wspec((1, HP)),
                  _wspec((HP, HP)), _wspec((1, HP)),
                  _wspec((HP, HP)), _wspec((1, HP)),
                  _wspec((HP, HP)), _wspec((HP, HP))],
        out_specs=[_rspec(NB, HP)] * 3,
        out_shape=[jax.ShapeDtypeStruct((N, HP), fdt),
                   jax.ShapeDtypeStruct((N, HP), fdt),
                   jax.ShapeDtypeStruct((N, HP), fdt)],
    )

    node_final = pl.pallas_call(
        _node_final_body,
        grid=ngrid,
        in_specs=[_wspec((1, HP)),
                  pl.BlockSpec((3, 2, NB, 128), lambda i: (0, 0, i, 0)),
                  _rspec(NB, HP),
                  _wspec((3, 128, HP)), _wspec((1, HP)),
                  _wspec((HP, HP)), _wspec((1, HP)),
                  _wspec((HP, HP)), _wspec((1, HP))],
        out_specs=_rspec(NB, H),
        out_shape=jax.ShapeDtypeStruct((N, H), fdt),
    )

    for d in range(3):
        g = gather(f_i, f_j, row, col)
        ed = edge_depth if d < 2 else edge_depth_final
        eh, m = ed(escale, eh, g,
                   wl, bl, em["W0"], em["b0"], em["W1"], em["b1"],
                   em["W2"], em["b2"], n1["W0"], n1["b0"], n1["W1"],
                   n1["b1"], n1["W2"], n1["b2"])
        agg = scatter(m, col, zeros_zr)
        if d < 2:
            xh, f_i, f_j = node_depth(nscale, agg, xh,
                                      n2w0g, n2["b0"], n2["W1"],
                                      n2["b1"], n2["W2"], n2["b2"], wi, wo)
        else:
            xh = node_final(nscale, agg, xh,
                            n2w0g, n2["b0"], n2["W1"], n2["b1"],
                            n2["W2"], n2["b2"])

    return xh, eh


# speedup vs baseline: 1.2607x; 1.0715x over previous
import functools

import jax
import jax.numpy as jnp
from jax import lax
from jax.experimental import pallas as pl
from jax.experimental.pallas import tpu as pltpu
from jax.experimental.pallas import tpu_sc as plsc

H = 300
HP = 384
HH = HP // 2
NC, NS = 2, 16
NW = NC * NS
CHUNK = 128

NB = 1000
EB = 1280


def _padw(w):
    pads = tuple((0, HP - d) if d == H else (0, 0) for d in w.shape)
    return jnp.pad(w, pads)


def _padb(b):
    return jnp.pad(b, ((0, HP - H),)).reshape(1, HP)


def _mm(a, b):
    return jnp.dot(a.astype(jnp.bfloat16), b.astype(jnp.bfloat16),
                   preferred_element_type=jnp.float32)


def _relu(a):
    return jnp.maximum(a, 0.0)


def _node_init_body(x_ref, w0, b0, w1, b1, w2, b2, wi, wo,
                    xh_out, fi_out, fj_out):
    t = _relu(_mm(x_ref[...], w0[...]) + b0[...])
    t = _relu(_mm(t, w1[...]) + b1[...])
    xh = _mm(t, w2[...]) + b2[...]
    xh_out[...] = xh
    fi_out[...] = _mm(xh, wi[...])
    fj_out[...] = _mm(xh, wo[...])


def _edge_init_body(ea_ref, w0, b0, w1, b1, w2, b2, eh_out):
    t = _relu(_mm(ea_ref[...], w0[...]) + b0[...])
    t = _relu(_mm(t, w1[...]) + b1[...])
    eh_out[...] = (_mm(t, w2[...]) + b2[...]).astype(jnp.bfloat16)


def _make_edge_depth_body(out_w):
    def body(scale_ref, eh_ref, g_ref,
             wl, bl, ew0, eb0, ew1, eb1, ew2, eb2,
             nw0, nb0, nw1, nb1, nw2, nb2,
             eh_out, m_out):
        eh = eh_ref[...]
        out = _relu(_mm(eh, wl[...]) + bl[...] + g_ref[...])
        t = _relu(_mm(out, ew0[...]) + eb0[...])
        t = _relu(_mm(t, ew1[...]) + eb1[...])
        eh_new = (scale_ref[...] * eh.astype(jnp.float32)
                  + _mm(t, ew2[...]) + eb2[...])
        eh_out[...] = eh_new[:, :out_w].astype(eh_out.dtype)
        u = _relu(_mm(eh_new, nw0[...]) + nb0[...])
        u = _relu(_mm(u, nw1[...]) + nb1[...])
        m = _mm(u, nw2[...]) + nb2[...]
        m_out[0] = m[:, 0:128]
        m_out[1] = m[:, 128:256]
        m_out[2] = m[:, 256:384]

    return body


def _agg_mlp_in(agg_ref, w0g, b0):
    acc = b0[...]
    for g in range(HP // 128):
        acc = acc + _mm(agg_ref[g, 0] + agg_ref[g, 1], w0g[g])
    return _relu(acc)


def _node_depth_body(scale_ref, agg_ref, xh_ref,
                     w0g, b0, w1, b1, w2, b2, wi, wo,
                     xh_out, fi_out, fj_out):
    t = _agg_mlp_in(agg_ref, w0g, b0)
    t = _relu(_mm(t, w1[...]) + b1[...])
    xh = scale_ref[...] * xh_ref[...] + _mm(t, w2[...]) + b2[...]
    xh_out[...] = xh
    fi_out[...] = _mm(xh, wi[...])
    fj_out[...] = _mm(xh, wo[...])


def _node_final_body(scale_ref, agg_ref, xh_ref,
                     w0g, b0, w1, b1, w2, b2, xh_out):
    t = _agg_mlp_in(agg_ref, w0g, b0)
    t = _relu(_mm(t, w1[...]) + b1[...])
    xh = scale_ref[...] * xh_ref[...] + _mm(t, w2[...]) + b2[...]
    xh_out[...] = xh[:, :H]


def _wspec(shape):
    nd = len(shape)
    return pl.BlockSpec(shape, lambda i: (0,) * nd)


def _rspec(rows, cols):
    return pl.BlockSpec((rows, cols), lambda i: (i, 0))


_MESH = plsc.VectorSubcoreMesh(core_axis_name="c", subcore_axis_name="s")


CG = 40


def _make_gather(E):
    per = E // NW
    nk = per // CG

    @functools.partial(
        pl.kernel,
        out_type=jax.ShapeDtypeStruct((E, HP), jnp.float32),
        mesh=_MESH,
        scratch_types=[pltpu.VMEM((per,), jnp.int32),
                       pltpu.VMEM((per,), jnp.int32),
                       pltpu.VMEM((CG, HP), jnp.float32),
                       pltpu.VMEM((CG, HP), jnp.float32),
                       pltpu.VMEM((CG, HP), jnp.float32),
                       pltpu.VMEM((CG, HP), jnp.float32),
                       pltpu.SemaphoreType.DMA,
                       pltpu.SemaphoreType.DMA,
                       pltpu.SemaphoreType.DMA,
                       pltpu.SemaphoreType.DMA],
    )
    def gather(fi_hbm, fj_hbm, row_hbm, col_hbm, g_hbm,
               idxa_v, idxb_v, ba0, ba1, bb0, bb1, sa0, sa1, sb0, sb1):
        wid = lax.axis_index("s") * NC + lax.axis_index("c")
        e0 = wid * per
        pltpu.sync_copy(row_hbm.at[pl.ds(e0, per)], idxa_v)
        pltpu.sync_copy(col_hbm.at[pl.ds(e0, per)], idxb_v)
        bufs = ((ba0, bb0, sa0, sb0), (ba1, bb1, sa1, sb1))

        def issue(k, p):
            ba, bb, sa, sb = bufs[p]
            s = pl.ds(k * CG, CG)
            pltpu.async_copy(fi_hbm.at[idxa_v.at[s]], ba, sa)
            pltpu.async_copy(fj_hbm.at[idxb_v.at[s]], bb, sb)

        def consume(k, p):
            ba, bb, sa, sb = bufs[p]
            s = pl.ds(k * CG, CG)
            pltpu.make_async_copy(fi_hbm.at[idxa_v.at[s]], ba, sa).wait()
            pltpu.make_async_copy(fj_hbm.at[idxb_v.at[s]], bb, sb).wait()

            def addrow(r, _):
                for j in range(HP // 16):
                    plsc.addupdate(ba.at[r, pl.ds(j * 16, 16)],
                                   bb[r, pl.ds(j * 16, 16)])
                return 0

            lax.fori_loop(0, CG, addrow, 0)
            pltpu.sync_copy(ba, g_hbm.at[pl.ds(e0 + k * CG, CG)])

        issue(0, 0)

        def step(kk, _):
            k0 = 2 * kk
            issue(k0 + 1, 1)
            consume(k0, 0)
            issue(k0 + 2, 0)
            consume(k0 + 1, 1)
            return 0

        lax.fori_loop(0, (nk - 1) // 2, step, 0)
        consume(nk - 1, 0)

    return gather


def _make_scatter(E, N):
    per = E // NW
    nk = per // CG
    ZR = 200
    ncp = N // ZR
    G = HP // 128

    @functools.partial(
        pl.kernel,
        out_type=jax.ShapeDtypeStruct((G, 2, N, 128), jnp.float32),
        mesh=_MESH,
        scratch_types=[pltpu.VMEM((CG,), jnp.int32),
                       pltpu.VMEM((CG,), jnp.int32),
                       pltpu.VMEM((CG, 128), jnp.float32),
                       pltpu.VMEM((CG, 128), jnp.float32),
                       pltpu.VMEM_SHARED((N, 128), jnp.float32),
                       pltpu.SemaphoreType.DMA,
                       pltpu.SemaphoreType.DMA,
                       pltpu.SemaphoreType.DMA,
                       pltpu.SemaphoreType.DMA],
    )
    def scatter(m_hbm, col_hbm, z_hbm, agg_hbm, ib0, ib1, mb0, mb1, acc_sh,
                si0, si1, sm0, sm1):
        cid = lax.axis_index("c")
        sid = lax.axis_index("s")
        e0 = (cid * NS + sid) * per
        mbufs = ((ib0, mb0, si0, sm0), (ib1, mb1, si1, sm1))

        for g in range(G):
            def zcp(k, _):
                j = sid + k * NS

                @pl.when(j < ncp)
                def _():
                    pltpu.sync_copy(z_hbm, acc_sh.at[pl.ds(j * ZR, ZR)])

                return 0

            lax.fori_loop(0, (ncp + NS - 1) // NS, zcp, 0)
            plsc.subcore_barrier()

            def issue(k, p):
                ib, mb, si, sm = mbufs[p]
                pltpu.async_copy(col_hbm.at[pl.ds(e0 + k * CG, CG)], ib, si)
                pltpu.async_copy(m_hbm.at[g, pl.ds(e0 + k * CG, CG)], mb, sm)

            def consume(k, p):
                ib, mb, si, sm = mbufs[p]
                pltpu.make_async_copy(
                    col_hbm.at[pl.ds(e0 + k * CG, CG)], ib, si).wait()
                pltpu.make_async_copy(
                    m_hbm.at[g, pl.ds(e0 + k * CG, CG)], mb, sm).wait()
                pltpu.sync_copy(mb, acc_sh.at[ib], add=True)

            issue(0, 0)

            def step(kk, _):
                k0 = 2 * kk
                issue(k0 + 1, 1)
                consume(k0, 0)
                issue(k0 + 2, 0)
                consume(k0 + 1, 1)
                return 0

            lax.fori_loop(0, (nk - 1) // 2, step, 0)
            consume(nk - 1, 0)
            plsc.subcore_barrier()

            def wcp(k, _):
                j = sid + k * NS

                @pl.when(j < ncp)
                def _():
                    pltpu.sync_copy(acc_sh.at[pl.ds(j * ZR, ZR)],
                                    agg_hbm.at[g, cid, pl.ds(j * ZR, ZR)])

                return 0

            lax.fori_loop(0, (ncp + NS - 1) // NS, wcp, 0)
            plsc.subcore_barrier()

    return scatter


def kernel(x, edge_index, edge_attr, params):
    N, ND = x.shape
    E, ED = edge_attr.shape
    row = edge_index[0]
    col = edge_index[1]
    zeros_zr = jnp.zeros((200, 128), jnp.float32)

    p = params
    escale = jnp.full((1, HP), 1.0 + p["edge_eps"][0], jnp.float32)
    nscale = jnp.full((1, HP), 1.0 + p["node_eps"][0], jnp.float32)

    ni = {k: _padw(v) if v.ndim == 2 else _padb(v)
          for k, v in p["node_init"].items()}
    ei = {k: _padw(v) if v.ndim == 2 else _padb(v)
          for k, v in p["edge_init"].items()}
    em = {k: _padw(v) if v.ndim == 2 else _padb(v)
          for k, v in p["edge_mlp"].items()}
    n1 = {k: _padw(v) if v.ndim == 2 else _padb(v)
          for k, v in p["node_mlp1"].items()}
    n2 = {k: _padw(v) if v.ndim == 2 else _padb(v)
          for k, v in p["node_mlp2"].items()}
    wl = _padw(p["edge_lin_W"])
    bl = _padb(p["edge_lin_b"])
    wi = _padw(p["node_in_W"])
    wo = _padw(p["node_out_W"])
    n2w0g = n2["W0"].reshape(HP // 128, 128, HP)

    fdt = jnp.float32
    ngrid = (N // NB,)
    egrid = (E // EB,)

    node_init = pl.pallas_call(
        _node_init_body,
        grid=ngrid,
        in_specs=[_rspec(NB, ND),
                  _wspec((ND, HP)), _wspec((1, HP)),
                  _wspec((HP, HP)), _wspec((1, HP)),
                  _wspec((HP, HP)), _wspec((1, HP)),
                  _wspec((HP, HP)), _wspec((HP, HP))],
        out_specs=[_rspec(NB, HP)] * 3,
        out_shape=[jax.ShapeDtypeStruct((N, HP), fdt),
                   jax.ShapeDtypeStruct((N, HP), fdt),
                   jax.ShapeDtypeStruct((N, HP), fdt)],
    )
    xh, f_i, f_j = node_init(x, ni["W0"], ni["b0"], ni["W1"], ni["b1"],
                             ni["W2"], ni["b2"], wi, wo)

    edge_init = pl.pallas_call(
        _edge_init_body,
        grid=egrid,
        in_specs=[_rspec(EB, ED),
                  _wspec((ED, HP)), _wspec((1, HP)),
                  _wspec((HP, HP)), _wspec((1, HP)),
                  _wspec((HP, HP)), _wspec((1, HP))],
        out_specs=_rspec(EB, HP),
        out_shape=jax.ShapeDtypeStruct((E, HP), jnp.bfloat16),
    )
    eh = edge_init(edge_attr, ei["W0"], ei["b0"], ei["W1"], ei["b1"],
                   ei["W2"], ei["b2"])

    gather = _make_gather(E)
    scatter = _make_scatter(E, N)

    edge_specs = dict(
        grid=egrid,
        in_specs=[_wspec((1, HP)),
                  _rspec(EB, HP), _rspec(EB, HP),
                  _wspec((HP, HP)), _wspec((1, HP)),
                  _wspec((HP, HP)), _wspec((1, HP)),
                  _wspec((HP, HP)), _wspec((1, HP)),
                  _wspec((HP, HP)), _wspec((1, HP)),
                  _wspec((HP, HP)), _wspec((1, HP)),
                  _wspec((HP, HP)), _wspec((1, HP)),
                  _wspec((HP, HP)), _wspec((1, HP))],
    )
    edge_depth = pl.pallas_call(
        _make_edge_depth_body(HP),
        out_specs=[_rspec(EB, HP),
                   pl.BlockSpec((3, EB, 128), lambda i: (0, i, 0))],
        out_shape=[jax.ShapeDtypeStruct((E, HP), jnp.bfloat16),
                   jax.ShapeDtypeStruct((3, E, 128), fdt)],
        **edge_specs,
    )
    edge_depth_final = pl.pallas_call(
        _make_edge_depth_body(H),
        out_specs=[_rspec(EB, H),
                   pl.BlockSpec((3, EB, 128), lambda i: (0, i, 0))],
        out_shape=[jax.ShapeDtypeStruct((E, H), fdt),
                   jax.ShapeDtypeStruct((3, E, 128), fdt)],
        **edge_specs,
    )

    node_depth = pl.pallas_call(
        _node_depth_body,
        grid=ngrid,
        in_specs=[_wspec((1, HP)),
                  pl.BlockSpec((3, 2, NB, 128), lambda i: (0, 0, i, 0)),
                  _rspec(NB, HP),
                  _wspec((3, 128, HP)), _wspec((1, HP)),
                  _wspec((HP, HP)), _wspec((1, HP)),
                  _wspec((HP, HP)), _wspec((1, HP)),
                  _wspec((HP, HP)), _wspec((HP, HP))],
        out_specs=[_rspec(NB, HP)] * 3,
        out_shape=[jax.ShapeDtypeStruct((N, HP), fdt),
                   jax.ShapeDtypeStruct((N, HP), fdt),
                   jax.ShapeDtypeStruct((N, HP), fdt)],
    )

    node_final = pl.pallas_call(
        _node_final_body,
        grid=ngrid,
        in_specs=[_wspec((1, HP)),
                  pl.BlockSpec((3, 2, NB, 128), lambda i: (0, 0, i, 0)),
                  _rspec(NB, HP),
                  _wspec((3, 128, HP)), _wspec((1, HP)),
                  _wspec((HP, HP)), _wspec((1, HP)),
                  _wspec((HP, HP)), _wspec((1, HP))],
        out_specs=_rspec(NB, H),
        out_shape=jax.ShapeDtypeStruct((N, H), fdt),
    )

    for d in range(3):
        g = gather(f_i, f_j, row, col)
        ed = edge_depth if d < 2 else edge_depth_final
        eh, m = ed(escale, eh, g,
                   wl, bl, em["W0"], em["b0"], em["W1"], em["b1"],
                   em["W2"], em["b2"], n1["W0"], n1["b0"], n1["W1"],
                   n1["b1"], n1["W2"], n1["b2"])
        agg = scatter(m, col, zeros_zr)
        if d < 2:
            xh, f_i, f_j = node_depth(nscale, agg, xh,
                                      n2w0g, n2["b0"], n2["W1"],
                                      n2["b1"], n2["W2"], n2["b2"], wi, wo)
        else:
            xh = node_final(nscale, agg, xh,
                            n2w0g, n2["b0"], n2["W1"], n2["b1"],
                            n2["W2"], n2["b2"])

    return xh, eh


# speedup vs baseline: 1.3521x; 1.0725x over previous
import functools

import jax
import jax.numpy as jnp
from jax import lax
from jax.experimental import pallas as pl
from jax.experimental.pallas import tpu as pltpu
from jax.experimental.pallas import tpu_sc as plsc

H = 300
HP = 384
HH = HP // 2
NC, NS = 2, 16
NW = NC * NS
CHUNK = 128

NB = 1000
EB = 1280


def _padw(w):
    pads = tuple((0, HP - d) if d == H else (0, 0) for d in w.shape)
    return jnp.pad(w, pads)


def _padb(b):
    return jnp.pad(b, ((0, HP - H),)).reshape(1, HP)


def _mm(a, b):
    return jnp.dot(a.astype(jnp.bfloat16), b.astype(jnp.bfloat16),
                   preferred_element_type=jnp.float32)


def _relu(a):
    return jnp.maximum(a, 0.0)


def _node_init_body(x_ref, w0, b0, w1, b1, w2, b2, wi, wo,
                    xh_out, fi_out, fj_out):
    t = _relu(_mm(x_ref[...], w0[...]) + b0[...])
    t = _relu(_mm(t, w1[...]) + b1[...])
    xh = _mm(t, w2[...]) + b2[...]
    xh_out[...] = xh
    fi_out[...] = _mm(xh, wi[...])
    fj_out[...] = _mm(xh, wo[...])


def _edge_init_body(ea_ref, w0, b0, w1, b1, w2, b2, eh_out):
    t = _relu(_mm(ea_ref[...], w0[...]) + b0[...])
    t = _relu(_mm(t, w1[...]) + b1[...])
    eh_out[...] = (_mm(t, w2[...]) + b2[...]).astype(jnp.bfloat16)


def _make_edge_depth_body(out_w):
    def body(scale_ref, eh_ref, g_ref,
             wl, bl, ew0, eb0, ew1, eb1, ew2, eb2,
             nw0, nb0, nw1, nb1, nw2, nb2,
             eh_out, m_out):
        eh = eh_ref[...]
        out = _relu(_mm(eh, wl[...]) + bl[...] + g_ref[...])
        t = _relu(_mm(out, ew0[...]) + eb0[...])
        t = _relu(_mm(t, ew1[...]) + eb1[...])
        eh_new = (scale_ref[...] * eh.astype(jnp.float32)
                  + _mm(t, ew2[...]) + eb2[...])
        eh_out[...] = eh_new[:, :out_w].astype(eh_out.dtype)
        u = _relu(_mm(eh_new, nw0[...]) + nb0[...])
        u = _relu(_mm(u, nw1[...]) + nb1[...])
        m = _mm(u, nw2[...]) + nb2[...]
        m_out[0] = m[:, 0:128]
        m_out[1] = m[:, 128:256]
        m_out[2] = m[:, 256:384]

    return body


def _agg_mlp_in(agga_ref, aggb_ref, w0g, b0):
    acc = b0[...]
    for g in range(HP // 128):
        parts = (agga_ref[g, 0] + agga_ref[g, 1]
                 + aggb_ref[g, 0] + aggb_ref[g, 1])
        acc = acc + _mm(parts, w0g[g])
    return _relu(acc)


def _node_depth_body(scale_ref, agga_ref, aggb_ref, xh_ref,
                     w0g, b0, w1, b1, w2, b2, wi, wo,
                     xh_out, fi_out, fj_out):
    t = _agg_mlp_in(agga_ref, aggb_ref, w0g, b0)
    t = _relu(_mm(t, w1[...]) + b1[...])
    xh = scale_ref[...] * xh_ref[...] + _mm(t, w2[...]) + b2[...]
    xh_out[...] = xh
    fi_out[...] = _mm(xh, wi[...])
    fj_out[...] = _mm(xh, wo[...])


def _node_final_body(scale_ref, agga_ref, aggb_ref, xh_ref,
                     w0g, b0, w1, b1, w2, b2, xh_out):
    t = _agg_mlp_in(agga_ref, aggb_ref, w0g, b0)
    t = _relu(_mm(t, w1[...]) + b1[...])
    xh = scale_ref[...] * xh_ref[...] + _mm(t, w2[...]) + b2[...]
    xh_out[...] = xh[:, :H]


def _wspec(shape):
    nd = len(shape)
    return pl.BlockSpec(shape, lambda i: (0,) * nd)


def _rspec(rows, cols):
    return pl.BlockSpec((rows, cols), lambda i: (i, 0))


_MESH = plsc.VectorSubcoreMesh(core_axis_name="c", subcore_axis_name="s")


CG = 40


def _pipeline(nk, issue, consume):
    issue(0, 0)

    def step(kk, _):
        k0 = 2 * kk
        issue(k0 + 1, 1)
        consume(k0, 0)
        issue(k0 + 2, 0)
        consume(k0 + 1, 1)
        return 0

    if nk % 2:
        lax.fori_loop(0, (nk - 1) // 2, step, 0)
        consume(nk - 1, 0)
    else:
        lax.fori_loop(0, (nk - 2) // 2, step, 0)
        issue(nk - 1, 1)
        consume(nk - 2, 0)
        consume(nk - 1, 1)


def _make_gather(goff, Eh):
    per = Eh // NW
    nk = per // CG

    @functools.partial(
        pl.kernel,
        out_type=jax.ShapeDtypeStruct((Eh, HP), jnp.float32),
        mesh=_MESH,
        scratch_types=[pltpu.VMEM((per,), jnp.int32),
                       pltpu.VMEM((per,), jnp.int32),
                       pltpu.VMEM((CG, HP), jnp.float32),
                       pltpu.VMEM((CG, HP), jnp.float32),
                       pltpu.VMEM((CG, HP), jnp.float32),
                       pltpu.VMEM((CG, HP), jnp.float32),
                       pltpu.SemaphoreType.DMA,
                       pltpu.SemaphoreType.DMA,
                       pltpu.SemaphoreType.DMA,
                       pltpu.SemaphoreType.DMA],
    )
    def gather(fi_hbm, fj_hbm, row_hbm, col_hbm, g_hbm,
               idxa_v, idxb_v, ba0, ba1, bb0, bb1, sa0, sa1, sb0, sb1):
        wid = lax.axis_index("s") * NC + lax.axis_index("c")
        e0 = wid * per
        pltpu.sync_copy(row_hbm.at[pl.ds(goff + e0, per)], idxa_v)
        pltpu.sync_copy(col_hbm.at[pl.ds(goff + e0, per)], idxb_v)
        bufs = ((ba0, bb0, sa0, sb0), (ba1, bb1, sa1, sb1))

        def issue(k, p):
            ba, bb, sa, sb = bufs[p]
            s = pl.ds(k * CG, CG)
            pltpu.async_copy(fi_hbm.at[idxa_v.at[s]], ba, sa)
            pltpu.async_copy(fj_hbm.at[idxb_v.at[s]], bb, sb)

        def consume(k, p):
            ba, bb, sa, sb = bufs[p]
            s = pl.ds(k * CG, CG)
            pltpu.make_async_copy(fi_hbm.at[idxa_v.at[s]], ba, sa).wait()
            pltpu.make_async_copy(fj_hbm.at[idxb_v.at[s]], bb, sb).wait()

            def addrow(r, _):
                for j in range(HP // 16):
                    plsc.addupdate(ba.at[r, pl.ds(j * 16, 16)],
                                   bb[r, pl.ds(j * 16, 16)])
                return 0

            lax.fori_loop(0, CG, addrow, 0)
            pltpu.sync_copy(ba, g_hbm.at[pl.ds(e0 + k * CG, CG)])

        _pipeline(nk, issue, consume)

    return gather


def _make_scatter(goff, Eh, N):
    per = Eh // NW
    nk = per // CG
    ZR = 200
    ncp = N // ZR
    G = HP // 128

    @functools.partial(
        pl.kernel,
        out_type=jax.ShapeDtypeStruct((G, 2, N, 128), jnp.float32),
        mesh=_MESH,
        scratch_types=[pltpu.VMEM((CG,), jnp.int32),
                       pltpu.VMEM((CG,), jnp.int32),
                       pltpu.VMEM((CG, 128), jnp.float32),
                       pltpu.VMEM((CG, 128), jnp.float32),
                       pltpu.VMEM_SHARED((N, 128), jnp.float32),
                       pltpu.SemaphoreType.DMA,
                       pltpu.SemaphoreType.DMA,
                       pltpu.SemaphoreType.DMA,
                       pltpu.SemaphoreType.DMA],
    )
    def scatter(m_hbm, col_hbm, z_hbm, agg_hbm, ib0, ib1, mb0, mb1, acc_sh,
                si0, si1, sm0, sm1):
        cid = lax.axis_index("c")
        sid = lax.axis_index("s")
        e0 = (cid * NS + sid) * per
        mbufs = ((ib0, mb0, si0, sm0), (ib1, mb1, si1, sm1))

        for g in range(G):
            def zcp(k, _):
                j = sid + k * NS

                @pl.when(j < ncp)
                def _():
                    pltpu.sync_copy(z_hbm, acc_sh.at[pl.ds(j * ZR, ZR)])

                return 0

            lax.fori_loop(0, (ncp + NS - 1) // NS, zcp, 0)
            plsc.subcore_barrier()

            def issue(k, p):
                ib, mb, si, sm = mbufs[p]
                pltpu.async_copy(
                    col_hbm.at[pl.ds(goff + e0 + k * CG, CG)], ib, si)
                pltpu.async_copy(m_hbm.at[g, pl.ds(e0 + k * CG, CG)], mb, sm)

            def consume(k, p):
                ib, mb, si, sm = mbufs[p]
                pltpu.make_async_copy(
                    col_hbm.at[pl.ds(goff + e0 + k * CG, CG)], ib, si).wait()
                pltpu.make_async_copy(
                    m_hbm.at[g, pl.ds(e0 + k * CG, CG)], mb, sm).wait()
                pltpu.sync_copy(mb, acc_sh.at[ib], add=True)

            _pipeline(nk, issue, consume)
            plsc.subcore_barrier()

            def wcp(k, _):
                j = sid + k * NS

                @pl.when(j < ncp)
                def _():
                    pltpu.sync_copy(acc_sh.at[pl.ds(j * ZR, ZR)],
                                    agg_hbm.at[g, cid, pl.ds(j * ZR, ZR)])

                return 0

            lax.fori_loop(0, (ncp + NS - 1) // NS, wcp, 0)
            plsc.subcore_barrier()

    return scatter


def kernel(x, edge_index, edge_attr, params):
    N, ND = x.shape
    E, ED = edge_attr.shape
    row = edge_index[0]
    col = edge_index[1]
    zeros_zr = jnp.zeros((200, 128), jnp.float32)

    p = params
    escale = jnp.full((1, HP), 1.0 + p["edge_eps"][0], jnp.float32)
    nscale = jnp.full((1, HP), 1.0 + p["node_eps"][0], jnp.float32)

    ni = {k: _padw(v) if v.ndim == 2 else _padb(v)
          for k, v in p["node_init"].items()}
    ei = {k: _padw(v) if v.ndim == 2 else _padb(v)
          for k, v in p["edge_init"].items()}
    em = {k: _padw(v) if v.ndim == 2 else _padb(v)
          for k, v in p["edge_mlp"].items()}
    n1 = {k: _padw(v) if v.ndim == 2 else _padb(v)
          for k, v in p["node_mlp1"].items()}
    n2 = {k: _padw(v) if v.ndim == 2 else _padb(v)
          for k, v in p["node_mlp2"].items()}
    wl = _padw(p["edge_lin_W"])
    bl = _padb(p["edge_lin_b"])
    wi = _padw(p["node_in_W"])
    wo = _padw(p["node_out_W"])
    n2w0g = n2["W0"].reshape(HP // 128, 128, HP)

    fdt = jnp.float32
    ngrid = (N // NB,)
    egrid = (E // EB,)

    node_init = pl.pallas_call(
        _node_init_body,
        grid=ngrid,
        in_specs=[_rspec(NB, ND),
                  _wspec((ND, HP)), _wspec((1, HP)),
                  _wspec((HP, HP)), _wspec((1, HP)),
                  _wspec((HP, HP)), _wspec((1, HP)),
                  _wspec((HP, HP)), _wspec((HP, HP))],
        out_specs=[_rspec(NB, HP)] * 3,
        out_shape=[jax.ShapeDtypeStruct((N, HP), fdt),
                   jax.ShapeDtypeStruct((N, HP), fdt),
                   jax.ShapeDtypeStruct((N, HP), fdt)],
    )
    xh, f_i, f_j = node_init(x, ni["W0"], ni["b0"], ni["W1"], ni["b1"],
                             ni["W2"], ni["b2"], wi, wo)

    EA = 81920
    EBH = E - EA

    def make_edge_init(Eh):
        return pl.pallas_call(
            _edge_init_body,
            grid=(Eh // EB,),
            in_specs=[_rspec(EB, ED),
                      _wspec((ED, HP)), _wspec((1, HP)),
                      _wspec((HP, HP)), _wspec((1, HP)),
                      _wspec((HP, HP)), _wspec((1, HP))],
            out_specs=_rspec(EB, HP),
            out_shape=jax.ShapeDtypeStruct((Eh, HP), jnp.bfloat16),
        )

    eiargs = (ei["W0"], ei["b0"], ei["W1"], ei["b1"], ei["W2"], ei["b2"])
    ehA = make_edge_init(EA)(edge_attr[:EA], *eiargs)
    ehB = make_edge_init(EBH)(edge_attr[EA:], *eiargs)

    gatherA = _make_gather(0, EA)
    gatherB = _make_gather(EA, EBH)
    scatterA = _make_scatter(0, EA, N)
    scatterB = _make_scatter(EA, EBH, N)

    def make_edge_depth(Eh, out_w):
        return pl.pallas_call(
            _make_edge_depth_body(out_w),
            grid=(Eh // EB,),
            in_specs=[_wspec((1, HP)),
                      _rspec(EB, HP), _rspec(EB, HP),
                      _wspec((HP, HP)), _wspec((1, HP)),
                      _wspec((HP, HP)), _wspec((1, HP)),
                      _wspec((HP, HP)), _wspec((1, HP)),
                      _wspec((HP, HP)), _wspec((1, HP)),
                      _wspec((HP, HP)), _wspec((1, HP)),
                      _wspec((HP, HP)), _wspec((1, HP)),
                      _wspec((HP, HP)), _wspec((1, HP))],
            out_specs=[_rspec(EB, out_w),
                       pl.BlockSpec((3, EB, 128), lambda i: (0, i, 0))],
            out_shape=[jax.ShapeDtypeStruct(
                           (Eh, out_w),
                           jnp.bfloat16 if out_w == HP else fdt),
                       jax.ShapeDtypeStruct((3, Eh, 128), fdt)],
        )

    edA, edB = make_edge_depth(EA, HP), make_edge_depth(EBH, HP)
    edAf, edBf = make_edge_depth(EA, H), make_edge_depth(EBH, H)
    ewargs = (wl, bl, em["W0"], em["b0"], em["W1"], em["b1"], em["W2"],
              em["b2"], n1["W0"], n1["b0"], n1["W1"], n1["b1"], n1["W2"],
              n1["b2"])

    aggspec = pl.BlockSpec((3, 2, NB, 128), lambda i: (0, 0, i, 0))
    node_depth = pl.pallas_call(
        _node_depth_body,
        grid=ngrid,
        in_specs=[_wspec((1, HP)), aggspec, aggspec,
                  _rspec(NB, HP),
                  _wspec((3, 128, HP)), _wspec((1, HP)),
                  _wspec((HP, HP)), _wspec((1, HP)),
                  _wspec((HP, HP)), _wspec((1, HP)),
                  _wspec((HP, HP)), _wspec((HP, HP))],
        out_specs=[_rspec(NB, HP)] * 3,
        out_shape=[jax.ShapeDtypeStruct((N, HP), fdt),
                   jax.ShapeDtypeStruct((N, HP), fdt),
                   jax.ShapeDtypeStruct((N, HP), fdt)],
    )

    node_final = pl.pallas_call(
        _node_final_body,
        grid=ngrid,
        in_specs=[_wspec((1, HP)), aggspec, aggspec,
                  _rspec(NB, HP),
                  _wspec((3, 128, HP)), _wspec((1, HP)),
                  _wspec((HP, HP)), _wspec((1, HP)),
                  _wspec((HP, HP)), _wspec((1, HP))],
        out_specs=_rspec(NB, H),
        out_shape=jax.ShapeDtypeStruct((N, H), fdt),
    )

    for d in range(3):
        gA = gatherA(f_i, f_j, row, col)
        f_i2, _ = lax.optimization_barrier((f_i, gA[:8, :8]))
        gB = gatherB(f_i2, f_j, row, col)
        ea, eb = (edA, edB) if d < 2 else (edAf, edBf)
        ehA, mA = ea(escale, ehA, gA, *ewargs)
        ehB, mB = eb(escale, ehB, gB, *ewargs)
        mA2, _ = lax.optimization_barrier((mA, gB[:8, :8]))
        aggA = scatterA(mA2, col, zeros_zr)
        mB2, _ = lax.optimization_barrier((mB, aggA[:, :, :8, :8]))
        aggB = scatterB(mB2, col, zeros_zr)
        if d < 2:
            xh, f_i, f_j = node_depth(nscale, aggA, aggB, xh,
                                      n2w0g, n2["b0"], n2["W1"],
                                      n2["b1"], n2["W2"], n2["b2"], wi, wo)
        else:
            xh = node_final(nscale, aggA, aggB, xh,
                            n2w0g, n2["b0"], n2["W1"], n2["b1"],
                            n2["W2"], n2["b2"])

    return xh, jnp.concatenate([ehA, ehB], axis=0)


# speedup vs baseline: 1.3617x; 1.0072x over previous
import functools

import jax
import jax.numpy as jnp
from jax import lax
from jax.experimental import pallas as pl
from jax.experimental.pallas import tpu as pltpu
from jax.experimental.pallas import tpu_sc as plsc

H = 300
HP = 384
HH = HP // 2
NC, NS = 2, 16
NW = NC * NS
CHUNK = 128

NB = 1000
EB = 1280


def _padw(w):
    pads = tuple((0, HP - d) if d == H else (0, 0) for d in w.shape)
    return jnp.pad(w, pads)


def _padb(b):
    return jnp.pad(b, ((0, HP - H),)).reshape(1, HP)


def _mm(a, b):
    return jnp.dot(a.astype(jnp.bfloat16), b.astype(jnp.bfloat16),
                   preferred_element_type=jnp.float32)


def _relu(a):
    return jnp.maximum(a, 0.0)


def _node_init_body(x_ref, w0, b0, w1, b1, w2, b2, wi, wo,
                    xh_out, fi_out, fj_out):
    t = _relu(_mm(x_ref[...], w0[...]) + b0[...])
    t = _relu(_mm(t, w1[...]) + b1[...])
    xh = _mm(t, w2[...]) + b2[...]
    xh_out[...] = xh
    fi_out[...] = _mm(xh, wi[...])
    fj_out[...] = _mm(xh, wo[...])


def _edge_init_body(ea_ref, w0, b0, w1, b1, w2, b2, eh_out):
    t = _relu(_mm(ea_ref[...], w0[...]) + b0[...])
    t = _relu(_mm(t, w1[...]) + b1[...])
    eh_out[...] = (_mm(t, w2[...]) + b2[...]).astype(jnp.bfloat16)


def _make_edge_depth_body(out_w):
    def body(scale_ref, eh_ref, g_ref,
             wl, bl, ew0, eb0, ew1, eb1, ew2, eb2,
             nw0, nb0, nw1, nb1, nw2, nb2,
             eh_out, m_out):
        eh = eh_ref[...]
        out = _relu(_mm(eh, wl[...]) + bl[...] + g_ref[...])
        t = _relu(_mm(out, ew0[...]) + eb0[...])
        t = _relu(_mm(t, ew1[...]) + eb1[...])
        eh_new = (scale_ref[...] * eh.astype(jnp.float32)
                  + _mm(t, ew2[...]) + eb2[...])
        eh_out[...] = eh_new[:, :out_w].astype(eh_out.dtype)
        u = _relu(_mm(eh_new, nw0[...]) + nb0[...])
        u = _relu(_mm(u, nw1[...]) + nb1[...])
        m = _mm(u, nw2[...]) + nb2[...]
        m_out[0] = m[:, 0:128]
        m_out[1] = m[:, 128:256]
        m_out[2] = m[:, 256:384]

    return body


def _agg_mlp_in(agga_ref, aggb_ref, w0g, b0):
    acc = b0[...]
    for g in range(HP // 128):
        parts = (agga_ref[g, 0] + agga_ref[g, 1]
                 + aggb_ref[g, 0] + aggb_ref[g, 1])
        acc = acc + _mm(parts, w0g[g])
    return _relu(acc)


def _node_depth_body(scale_ref, agga_ref, aggb_ref, xh_ref,
                     w0g, b0, w1, b1, w2, b2, wi, wo,
                     xh_out, fi_out, fj_out):
    t = _agg_mlp_in(agga_ref, aggb_ref, w0g, b0)
    t = _relu(_mm(t, w1[...]) + b1[...])
    xh = scale_ref[...] * xh_ref[...] + _mm(t, w2[...]) + b2[...]
    xh_out[...] = xh
    fi_out[...] = _mm(xh, wi[...])
    fj_out[...] = _mm(xh, wo[...])


def _node_final_body(scale_ref, agga_ref, aggb_ref, xh_ref,
                     w0g, b0, w1, b1, w2, b2, xh_out):
    t = _agg_mlp_in(agga_ref, aggb_ref, w0g, b0)
    t = _relu(_mm(t, w1[...]) + b1[...])
    xh = scale_ref[...] * xh_ref[...] + _mm(t, w2[...]) + b2[...]
    xh_out[...] = xh[:, :H]


def _wspec(shape):
    nd = len(shape)
    return pl.BlockSpec(shape, lambda i: (0,) * nd)


def _rspec(rows, cols):
    return pl.BlockSpec((rows, cols), lambda i: (i, 0))


_MESH = plsc.VectorSubcoreMesh(core_axis_name="c", subcore_axis_name="s")


CG = 40


def _pipeline(nk, issue, consume):
    issue(0, 0)

    def step(kk, _):
        k0 = 2 * kk
        issue(k0 + 1, 1)
        consume(k0, 0)
        issue(k0 + 2, 0)
        consume(k0 + 1, 1)
        return 0

    if nk % 2:
        lax.fori_loop(0, (nk - 1) // 2, step, 0)
        consume(nk - 1, 0)
    else:
        lax.fori_loop(0, (nk - 2) // 2, step, 0)
        issue(nk - 1, 1)
        consume(nk - 2, 0)
        consume(nk - 1, 1)


def _make_gather(goff, Eh):
    per = Eh // NW
    nk = per // CG

    @functools.partial(
        pl.kernel,
        out_type=jax.ShapeDtypeStruct((Eh, HP), jnp.float32),
        mesh=_MESH,
        scratch_types=[pltpu.VMEM((per,), jnp.int32),
                       pltpu.VMEM((per,), jnp.int32),
                       pltpu.VMEM((CG, HP), jnp.float32),
                       pltpu.VMEM((CG, HP), jnp.float32),
                       pltpu.VMEM((CG, HP), jnp.float32),
                       pltpu.VMEM((CG, HP), jnp.float32),
                       pltpu.SemaphoreType.DMA,
                       pltpu.SemaphoreType.DMA,
                       pltpu.SemaphoreType.DMA,
                       pltpu.SemaphoreType.DMA],
    )
    def gather(fi_hbm, fj_hbm, row_hbm, col_hbm, g_hbm,
               idxa_v, idxb_v, ba0, ba1, bb0, bb1, sa0, sa1, sb0, sb1):
        wid = lax.axis_index("s") * NC + lax.axis_index("c")
        e0 = wid * per
        pltpu.sync_copy(row_hbm.at[pl.ds(goff + e0, per)], idxa_v)
        pltpu.sync_copy(col_hbm.at[pl.ds(goff + e0, per)], idxb_v)
        bufs = ((ba0, bb0, sa0, sb0), (ba1, bb1, sa1, sb1))

        def issue(k, p):
            ba, bb, sa, sb = bufs[p]
            s = pl.ds(k * CG, CG)
            pltpu.async_copy(fi_hbm.at[idxa_v.at[s]], ba, sa)
            pltpu.async_copy(fj_hbm.at[idxb_v.at[s]], bb, sb)

        def consume(k, p):
            ba, bb, sa, sb = bufs[p]
            s = pl.ds(k * CG, CG)
            pltpu.make_async_copy(fi_hbm.at[idxa_v.at[s]], ba, sa).wait()
            pltpu.make_async_copy(fj_hbm.at[idxb_v.at[s]], bb, sb).wait()

            def addrow(r, _):
                for j in range(HP // 16):
                    plsc.addupdate(ba.at[r, pl.ds(j * 16, 16)],
                                   bb[r, pl.ds(j * 16, 16)])
                return 0

            lax.fori_loop(0, CG, addrow, 0)
            pltpu.sync_copy(ba, g_hbm.at[pl.ds(e0 + k * CG, CG)])

        _pipeline(nk, issue, consume)

    return gather


def _make_scatter(goff, Eh, N):
    per = Eh // NW
    nk = per // CG
    ZR = 200
    ncp = N // ZR
    G = HP // 128

    @functools.partial(
        pl.kernel,
        out_type=jax.ShapeDtypeStruct((G, 2, N, 128), jnp.float32),
        mesh=_MESH,
        scratch_types=[pltpu.VMEM((nk, CG), jnp.int32),
                       pltpu.VMEM((CG, 128), jnp.float32),
                       pltpu.VMEM((CG, 128), jnp.float32),
                       pltpu.VMEM_SHARED((N, 128), jnp.float32),
                       pltpu.SemaphoreType.DMA,
                       pltpu.SemaphoreType.DMA],
    )
    def scatter(m_hbm, col3_hbm, z_hbm, agg_hbm, idx_v, mb0, mb1, acc_sh,
                sm0, sm1):
        cid = lax.axis_index("c")
        sid = lax.axis_index("s")
        tid = cid * NS + sid
        e0 = tid * per
        pltpu.sync_copy(col3_hbm.at[tid], idx_v)
        mbufs = ((mb0, sm0), (mb1, sm1))

        for g in range(G):
            def zcp(k, _):
                j = sid + k * NS

                @pl.when(j < ncp)
                def _():
                    pltpu.sync_copy(z_hbm, acc_sh.at[pl.ds(j * ZR, ZR)])

                return 0

            lax.fori_loop(0, (ncp + NS - 1) // NS, zcp, 0)
            plsc.subcore_barrier()

            def issue(k, p):
                mb, sm = mbufs[p]
                pltpu.async_copy(m_hbm.at[g, pl.ds(e0 + k * CG, CG)], mb, sm)

            def consume(k, p):
                mb, sm = mbufs[p]
                pltpu.make_async_copy(
                    m_hbm.at[g, pl.ds(e0 + k * CG, CG)], mb, sm).wait()
                pltpu.sync_copy(mb, acc_sh.at[idx_v.at[k]], add=True)

            _pipeline(nk, issue, consume)
            plsc.subcore_barrier()

            def wcp(k, _):
                j = sid + k * NS

                @pl.when(j < ncp)
                def _():
                    pltpu.sync_copy(acc_sh.at[pl.ds(j * ZR, ZR)],
                                    agg_hbm.at[g, cid, pl.ds(j * ZR, ZR)])

                return 0

            lax.fori_loop(0, (ncp + NS - 1) // NS, wcp, 0)
            plsc.subcore_barrier()

    return scatter


def kernel(x, edge_index, edge_attr, params):
    N, ND = x.shape
    E, ED = edge_attr.shape
    row = edge_index[0]
    col = edge_index[1]
    zeros_zr = jnp.zeros((200, 128), jnp.float32)

    p = params
    escale = jnp.full((1, HP), 1.0 + p["edge_eps"][0], jnp.float32)
    nscale = jnp.full((1, HP), 1.0 + p["node_eps"][0], jnp.float32)

    ni = {k: _padw(v) if v.ndim == 2 else _padb(v)
          for k, v in p["node_init"].items()}
    ei = {k: _padw(v) if v.ndim == 2 else _padb(v)
          for k, v in p["edge_init"].items()}
    em = {k: _padw(v) if v.ndim == 2 else _padb(v)
          for k, v in p["edge_mlp"].items()}
    n1 = {k: _padw(v) if v.ndim == 2 else _padb(v)
          for k, v in p["node_mlp1"].items()}
    n2 = {k: _padw(v) if v.ndim == 2 else _padb(v)
          for k, v in p["node_mlp2"].items()}
    wl = _padw(p["edge_lin_W"])
    bl = _padb(p["edge_lin_b"])
    wi = _padw(p["node_in_W"])
    wo = _padw(p["node_out_W"])
    n2w0g = n2["W0"].reshape(HP // 128, 128, HP)

    fdt = jnp.float32
    ngrid = (N // NB,)
    egrid = (E // EB,)

    node_init = pl.pallas_call(
        _node_init_body,
        grid=ngrid,
        in_specs=[_rspec(NB, ND),
                  _wspec((ND, HP)), _wspec((1, HP)),
                  _wspec((HP, HP)), _wspec((1, HP)),
                  _wspec((HP, HP)), _wspec((1, HP)),
                  _wspec((HP, HP)), _wspec((HP, HP))],
        out_specs=[_rspec(NB, HP)] * 3,
        out_shape=[jax.ShapeDtypeStruct((N, HP), fdt),
                   jax.ShapeDtypeStruct((N, HP), fdt),
                   jax.ShapeDtypeStruct((N, HP), fdt)],
    )
    xh, f_i, f_j = node_init(x, ni["W0"], ni["b0"], ni["W1"], ni["b1"],
                             ni["W2"], ni["b2"], wi, wo)

    EA = 81920
    EBH = E - EA

    def make_edge_init(Eh):
        return pl.pallas_call(
            _edge_init_body,
            grid=(Eh // EB,),
            in_specs=[_rspec(EB, ED),
                      _wspec((ED, HP)), _wspec((1, HP)),
                      _wspec((HP, HP)), _wspec((1, HP)),
                      _wspec((HP, HP)), _wspec((1, HP))],
            out_specs=_rspec(EB, HP),
            out_shape=jax.ShapeDtypeStruct((Eh, HP), jnp.bfloat16),
        )

    eiargs = (ei["W0"], ei["b0"], ei["W1"], ei["b1"], ei["W2"], ei["b2"])
    ehA = make_edge_init(EA)(edge_attr[:EA], *eiargs)
    ehB = make_edge_init(EBH)(edge_attr[EA:], *eiargs)

    col3A = col[:EA].reshape(NW, EA // (NW * CG), CG)
    col3B = col[EA:].reshape(NW, EBH // (NW * CG), CG)
    gatherA = _make_gather(0, EA)
    gatherB = _make_gather(EA, EBH)
    scatterA = _make_scatter(0, EA, N)
    scatterB = _make_scatter(EA, EBH, N)

    def make_edge_depth(Eh, out_w):
        return pl.pallas_call(
            _make_edge_depth_body(out_w),
            grid=(Eh // EB,),
            in_specs=[_wspec((1, HP)),
                      _rspec(EB, HP), _rspec(EB, HP),
                      _wspec((HP, HP)), _wspec((1, HP)),
                      _wspec((HP, HP)), _wspec((1, HP)),
                      _wspec((HP, HP)), _wspec((1, HP)),
                      _wspec((HP, HP)), _wspec((1, HP)),
                      _wspec((HP, HP)), _wspec((1, HP)),
                      _wspec((HP, HP)), _wspec((1, HP)),
                      _wspec((HP, HP)), _wspec((1, HP))],
            out_specs=[_rspec(EB, out_w),
                       pl.BlockSpec((3, EB, 128), lambda i: (0, i, 0))],
            out_shape=[jax.ShapeDtypeStruct(
                           (Eh, out_w),
                           jnp.bfloat16 if out_w == HP else fdt),
                       jax.ShapeDtypeStruct((3, Eh, 128), fdt)],
        )

    edA, edB = make_edge_depth(EA, HP), make_edge_depth(EBH, HP)
    edAf, edBf = make_edge_depth(EA, H), make_edge_depth(EBH, H)
    ewargs = (wl, bl, em["W0"], em["b0"], em["W1"], em["b1"], em["W2"],
              em["b2"], n1["W0"], n1["b0"], n1["W1"], n1["b1"], n1["W2"],
              n1["b2"])

    aggspec = pl.BlockSpec((3, 2, NB, 128), lambda i: (0, 0, i, 0))
    node_depth = pl.pallas_call(
        _node_depth_body,
        grid=ngrid,
        in_specs=[_wspec((1, HP)), aggspec, aggspec,
                  _rspec(NB, HP),
                  _wspec((3, 128, HP)), _wspec((1, HP)),
                  _wspec((HP, HP)), _wspec((1, HP)),
                  _wspec((HP, HP)), _wspec((1, HP)),
                  _wspec((HP, HP)), _wspec((HP, HP))],
        out_specs=[_rspec(NB, HP)] * 3,
        out_shape=[jax.ShapeDtypeStruct((N, HP), fdt),
                   jax.ShapeDtypeStruct((N, HP), fdt),
                   jax.ShapeDtypeStruct((N, HP), fdt)],
    )

    node_final = pl.pallas_call(
        _node_final_body,
        grid=ngrid,
        in_specs=[_wspec((1, HP)), aggspec, aggspec,
                  _rspec(NB, HP),
                  _wspec((3, 128, HP)), _wspec((1, HP)),
                  _wspec((HP, HP)), _wspec((1, HP)),
                  _wspec((HP, HP)), _wspec((1, HP))],
        out_specs=_rspec(NB, H),
        out_shape=jax.ShapeDtypeStruct((N, H), fdt),
    )

    for d in range(3):
        gA = gatherA(f_i, f_j, row, col)
        f_i2, _ = lax.optimization_barrier((f_i, gA[:8, :8]))
        gB = gatherB(f_i2, f_j, row, col)
        ea, eb = (edA, edB) if d < 2 else (edAf, edBf)
        ehA, mA = ea(escale, ehA, gA, *ewargs)
        ehB, mB = eb(escale, ehB, gB, *ewargs)
        mA2, _ = lax.optimization_barrier((mA, gB[:8, :8]))
        aggA = scatterA(mA2, col3A, zeros_zr)
        mB2, _ = lax.optimization_barrier((mB, aggA[:, :, :8, :8]))
        aggB = scatterB(mB2, col3B, zeros_zr)
        if d < 2:
            xh, f_i, f_j = node_depth(nscale, aggA, aggB, xh,
                                      n2w0g, n2["b0"], n2["W1"],
                                      n2["b1"], n2["W2"], n2["b2"], wi, wo)
        else:
            xh = node_final(nscale, aggA, aggB, xh,
                            n2w0g, n2["b0"], n2["W1"], n2["b1"],
                            n2["W2"], n2["b2"])

    return xh, jnp.concatenate([ehA, ehB], axis=0)
